# baseline (speedup 1.0000x reference)
import functools

import jax
import jax.numpy as jnp
from jax import lax
from jax.experimental import pallas as pl
from jax.experimental.pallas import tpu as pltpu
from jax.experimental.pallas import tpu_sc as plsc

B = 4096
S = 200
D = 64
H = 64
NCLS = 4
NCLS_PAD = 8

NC = 2
NS = 16
NW = NC * NS
L = 16

B_PER_W = B // NW
C0, C1 = 104, 96
NBUF = 4
LANE_CHUNKS = D // L
ACC_UNROLL = 8

_mesh = plsc.VectorSubcoreMesh(core_axis_name="c", subcore_axis_name="s")


@functools.partial(
    pl.kernel,
    mesh=_mesh,
    compiler_params=pltpu.CompilerParams(use_tc_tiling_on_sc=False),
    out_type=jax.ShapeDtypeStruct((B, D), jnp.float32),
    scratch_types=[
        pltpu.VMEM((B_PER_W * S,), jnp.int32),
        *[pltpu.VMEM((S, D), jnp.float32) for _ in range(NBUF)],
        pltpu.VMEM((B_PER_W, D), jnp.float32),
        *[pltpu.SemaphoreType.DMA for _ in range(NBUF)],
    ],
)
def _pool_sums(x_hbm, emb_hbm, out_hbm, idx_v, *rest):
    bufs = rest[:NBUF]
    out_v = rest[NBUF]
    sems = rest[NBUF + 1 : NBUF + 1 + NBUF]

    wid = lax.axis_index("s") * NC + lax.axis_index("c")
    base = wid * B_PER_W

    pltpu.sync_copy(x_hbm.at[pl.ds(base * S, B_PER_W * S)], idx_v)

    def issue(row, buf, sem):
        pltpu.async_copy(emb_hbm.at[idx_v.at[pl.ds(row * S, C0)]],
                         buf.at[pl.ds(0, C0)], sem)
        pltpu.async_copy(emb_hbm.at[idx_v.at[pl.ds(row * S + C0, C1)]],
                         buf.at[pl.ds(C0, C1)], sem)

    def wait(row, buf, sem):
        pltpu.make_async_copy(emb_hbm.at[idx_v.at[pl.ds(row * S, C0)]],
                              buf.at[pl.ds(0, C0)], sem).wait()
        pltpu.make_async_copy(emb_hbm.at[idx_v.at[pl.ds(row * S + C0, C1)]],
                              buf.at[pl.ds(C0, C1)], sem).wait()

    def reduce_row(row, buf):
        def acc_body(j8, accs):
            new = list(accs)
            for jj in range(ACC_UNROLL):
                j = j8 * ACC_UNROLL + jj
                for k in range(LANE_CHUNKS):
                    new[k] = new[k] + buf[j, pl.ds(k * L, L)]
            return tuple(new)

        accs = tuple(jnp.zeros((L,), jnp.float32) for _ in range(LANE_CHUNKS))
        accs = lax.fori_loop(0, S // ACC_UNROLL, acc_body, accs)
        for k in range(LANE_CHUNKS):
            out_v[row, pl.ds(k * L, L)] = accs[k]

    for p in range(NBUF):
        issue(p, bufs[p], sems[p])

    def body(blk, _):
        for p in range(NBUF):
            row = blk * NBUF + p
            wait(row, bufs[p], sems[p])
            reduce_row(row, bufs[p])
            issue(row + NBUF, bufs[p], sems[p])
        return ()

    lax.fori_loop(0, B_PER_W // NBUF - 1, body, ())

    for p in range(NBUF):
        row = B_PER_W - NBUF + p
        wait(row, bufs[p], sems[p])
        reduce_row(row, bufs[p])

    pltpu.sync_copy(out_v, out_hbm.at[pl.ds(base, B_PER_W)])


def _mlp_body(sums_ref, w1t_ref, b1_ref, w2t_ref, b2_ref, out_ref):
    pooled = sums_ref[...] * (1.0 / S)
    h = jnp.dot(pooled, w1t_ref[...], preferred_element_type=jnp.float32)
    h = jnp.maximum(h + b1_ref[...], 0.0)
    out_ref[...] = (
        jnp.dot(h, w2t_ref[...], preferred_element_type=jnp.float32)
        + b2_ref[...]
    )


_mlp = pl.pallas_call(
    _mlp_body,
    out_shape=jax.ShapeDtypeStruct((B, NCLS_PAD), jnp.float32),
)


def kernel(x, emb, W1, b1, W2, b2):
    x32 = x.astype(jnp.int32).reshape(-1)
    sums = _pool_sums(x32, emb)
    w2t_pad = jnp.zeros((H, NCLS_PAD), jnp.float32).at[:, :NCLS].set(W2.T)
    b2_pad = jnp.zeros((1, NCLS_PAD), jnp.float32).at[0, :NCLS].set(b2)
    logits = _mlp(sums, W1.T, b1.reshape(1, H), w2t_pad, b2_pad)
    return logits[:, :NCLS]
